# baseline (speedup 1.0000x reference)
import functools

import jax
import jax.numpy as jnp
from jax import lax
from jax.experimental import pallas as pl
from jax.experimental.pallas import tpu as pltpu
from jax.experimental.pallas import tpu_sc as plsc

NCORE = 2
NS = 16
LANES = 16
EB = 40
RBLK = 1000

_SC_PARAMS = pltpu.CompilerParams(
    needs_layout_passes=False, use_tc_tiling_on_sc=False)


def _degree_kernel(N, E):
    per_tec = E // NS
    n_vec = per_tec // LANES
    mesh = plsc.VectorSubcoreMesh(core_axis_name="c", subcore_axis_name="s")

    @functools.partial(
        pl.kernel,
        mesh=mesh,
        out_type=jax.ShapeDtypeStruct((NCORE, NS, N), jnp.float32),
        compiler_params=_SC_PARAMS,
        scratch_types=[
            pltpu.VMEM((per_tec,), jnp.int32),
            pltpu.VMEM((N,), jnp.float32),
        ],
    )
    def deg_k(ei, out, ibuf, hist):
        cid = lax.axis_index("c")
        sid = lax.axis_index("s")
        pltpu.sync_copy(ei.at[cid].at[sid], ibuf)

        z16 = jnp.zeros((LANES,), jnp.float32)

        def zero_body(i, _):
            hist[pl.ds(i * LANES, LANES)] = z16
            return 0

        lax.fori_loop(0, N // LANES, zero_body, 0)

        ones16 = jnp.ones((LANES,), jnp.float32)

        def hist_body(i, _):
            idx = ibuf[pl.ds(i * LANES, LANES)]
            plsc.addupdate_scatter(hist, [idx], ones16)
            return 0

        lax.fori_loop(0, n_vec, hist_body, 0)
        pltpu.sync_copy(hist, out.at[cid].at[sid])

    return deg_k


def _spmm_kernel(N, E, C, Dc, R):
    per_tec = E // NS
    nb = per_tec // EB
    G = nb // R
    assert G % 2 == 0 and G * R == nb
    cpc = C // NCORE
    rows_per_tile = N // NS
    ZR = 25
    nzc = rows_per_tile // ZR
    mesh = plsc.VectorSubcoreMesh(core_axis_name="c", subcore_axis_name="s")

    scratch = []
    for _p in range(2):
        for _b in range(R):
            scratch.append(pltpu.VMEM((EB, Dc), jnp.float32))
    for _p in range(2):
        scratch.append(pltpu.VMEM((R, EB), jnp.int32))
        scratch.append(pltpu.VMEM((R, EB), jnp.int32))
    scratch.append(pltpu.VMEM((ZR, Dc), jnp.float32))
    scratch.append(pltpu.VMEM_SHARED((N, Dc), jnp.float32))
    for _p in range(2):
        scratch.append(pltpu.SemaphoreType.DMA)
    for _p in range(2):
        scratch.append(pltpu.SemaphoreType.DMA)

    @functools.partial(
        pl.kernel,
        mesh=mesh,
        out_type=jax.ShapeDtypeStruct((C, N, Dc), jnp.float32),
        compiler_params=_SC_PARAMS,
        scratch_types=scratch,
    )
    def spmm_k(tab, srcT, dstT, out, *scr):
        rows = [[scr[p * R + b] for b in range(R)] for p in range(2)]
        base = 2 * R
        sbuf = [scr[base + 2 * p] for p in range(2)]
        dbuf = [scr[base + 2 * p + 1] for p in range(2)]
        zrow = scr[base + 4]
        acc = scr[base + 5]
        gsem = [scr[base + 6 + p] for p in range(2)]
        ssem = [scr[base + 8 + p] for p in range(2)]

        cid = lax.axis_index("c")
        sid = lax.axis_index("s")

        z16 = jnp.zeros((LANES,), jnp.float32)

        def zfill(r, _):
            for t in range(Dc // LANES):
                zrow[r, pl.ds(t * LANES, LANES)] = z16
            return 0

        lax.fori_loop(0, ZR, zfill, 0)

        row0 = sid * rows_per_tile

        def fire_gathers(p):
            for b in range(R):
                pltpu.async_copy(tab.at[sbuf[p].at[b]], rows[p][b], gsem[p])

        def wait_gathers(p):
            for b in range(R):
                pltpu.make_async_copy(
                    tab.at[sbuf[p].at[b]], rows[p][b], gsem[p]).wait()

        def fire_scatters(p):
            for b in range(R):
                pltpu.async_copy(
                    rows[p][b], acc.at[dbuf[p].at[b]], ssem[p], add=True)

        def wait_scatters(p):
            for b in range(R):
                pltpu.make_async_copy(
                    rows[p][b], acc.at[dbuf[p].at[b]], ssem[p]).wait()

        for ci in range(cpc):
            k = cid + NCORE * ci
            for j in range(nzc):
                pltpu.sync_copy(zrow, acc.at[pl.ds(row0 + j * ZR, ZR)])
            plsc.subcore_barrier()
            srck = srcT.at[k].at[sid]
            dstk = dstT.at[sid]

            pltpu.sync_copy(srck.at[0], sbuf[0])
            fire_gathers(0)

            def outer_pair(h, _):
                g0 = 2 * h
                wait_gathers(0)
                pltpu.sync_copy(dstk.at[g0], dbuf[0])
                fire_scatters(0)
                pltpu.sync_copy(srck.at[g0 + 1], sbuf[1])

                @pl.when(h >= 1)
                def _():
                    wait_scatters(1)

                fire_gathers(1)
                wait_gathers(1)
                pltpu.sync_copy(dstk.at[g0 + 1], dbuf[1])
                fire_scatters(1)
                wait_scatters(0)

                @pl.when(h < G // 2 - 1)
                def _():
                    pltpu.sync_copy(srck.at[g0 + 2], sbuf[0])
                    fire_gathers(0)

                return 0

            lax.fori_loop(0, G // 2, outer_pair, 0)
            wait_scatters(1)

            plsc.subcore_barrier()
            for j in range(nzc):
                sl = pl.ds(row0 + j * ZR, ZR)
                pltpu.sync_copy(acc.at[sl], out.at[k].at[sl])

    return spmm_k


def _prescale(dp_t, emb, N):
    nblk = N // RBLK

    def body(dp_ref, emb_ref, h1_ref, nrm_ref):
        deg = jnp.sum(dp_ref[...], axis=2)
        nrm = jnp.where(deg > 0, lax.rsqrt(deg), 0.0)
        nrm_ref[...] = nrm
        h = emb_ref[...] * nrm[:, 0:1]
        h1_ref[0] = h[:, :160]
        h1_ref[1] = jnp.concatenate(
            [h[:, 160:300], jnp.zeros((RBLK, 20), jnp.float32)], axis=1)

    return pl.pallas_call(
        body,
        grid=(nblk,),
        in_specs=[
            pl.BlockSpec((RBLK, NCORE, NS), lambda i: (i, 0, 0)),
            pl.BlockSpec((RBLK, 300), lambda i: (i, 0)),
        ],
        out_specs=[
            pl.BlockSpec((2, RBLK, 160), lambda i: (0, i, 0)),
            pl.BlockSpec((RBLK, 2), lambda i: (i, 0)),
        ],
        out_shape=[
            jax.ShapeDtypeStruct((2, N, 160), jnp.float32),
            jax.ShapeDtypeStruct((N, 2), jnp.float32),
        ],
    )(dp_t, emb)


def _layer1(agg1, nrm, W1p, b1, N):
    nblk = N // RBLK

    def body(a_ref, n_ref, w_ref, b_ref, o_ref):
        x = jnp.concatenate([a_ref[0], a_ref[1]], axis=1)
        nb2 = n_ref[...]
        x = x * nb2[:, 1:2]
        y = jnp.dot(x, w_ref[...], preferred_element_type=jnp.float32)
        y = y + b_ref[...]
        y = jnp.where(y >= 0.0, y, 0.2 * y)
        y = y * nb2[:, 0:1]
        for kk in range(3):
            o_ref[kk] = y[:, 112 * kk:112 * (kk + 1)]
        o_ref[3] = jnp.concatenate(
            [y[:, 336:400], jnp.zeros((RBLK, 48), jnp.float32)], axis=1)

    return pl.pallas_call(
        body,
        grid=(nblk,),
        in_specs=[
            pl.BlockSpec((2, RBLK, 160), lambda i: (0, i, 0)),
            pl.BlockSpec((RBLK, 2), lambda i: (i, 0)),
            pl.BlockSpec((320, 400), lambda i: (0, 0)),
            pl.BlockSpec((1, 400), lambda i: (0, 0)),
        ],
        out_specs=pl.BlockSpec((4, RBLK, 112), lambda i: (0, i, 0)),
        out_shape=jax.ShapeDtypeStruct((4, N, 112), jnp.float32),
    )(agg1, nrm, W1p, b1)


def _layer2(agg2, nrm, W2p, b2, N):
    nblk = N // RBLK

    def body(a_ref, n_ref, w_ref, b_ref, o_ref):
        x = jnp.concatenate([a_ref[kk] for kk in range(4)], axis=1)
        x = x * n_ref[:, 1:2]
        o_ref[...] = (
            jnp.dot(x, w_ref[...], preferred_element_type=jnp.float32)
            + b_ref[...])

    return pl.pallas_call(
        body,
        grid=(nblk,),
        in_specs=[
            pl.BlockSpec((4, RBLK, 112), lambda i: (0, i, 0)),
            pl.BlockSpec((RBLK, 2), lambda i: (i, 0)),
            pl.BlockSpec((448, 512), lambda i: (0, 0)),
            pl.BlockSpec((1, 512), lambda i: (0, 0)),
        ],
        out_specs=pl.BlockSpec((RBLK, 512), lambda i: (i, 0)),
        out_shape=jax.ShapeDtypeStruct((N, 512), jnp.float32),
    )(agg2, nrm, W2p, b2)


def kernel(init_emb, W1, b1, W2, b2, edge_index):
    N = init_emb.shape[0]
    E = edge_index.shape[1]

    per_tec = E // NS
    nb = per_tec // EB
    ei3d = edge_index.reshape(2, NS, per_tec)

    def edge_tables(C, R):
        G = nb // R
        src = edge_index[0].reshape(1, NS, G, R, EB)
        dst = edge_index[1].reshape(NS, G, R, EB)
        offs = (jnp.arange(C, dtype=jnp.int32) * N).reshape(C, 1, 1, 1, 1)
        return src + offs, dst

    W1p = jnp.concatenate([W1, jnp.zeros((20, 400), jnp.float32)], axis=0)
    W2p = jnp.concatenate([W2, jnp.zeros((48, 512), jnp.float32)], axis=0)
    b1r = b1.reshape(1, 400)
    b2r = b2.reshape(1, 512)

    deg_parts = _degree_kernel(N, E)(ei3d)
    dp_t = jnp.transpose(deg_parts, (2, 0, 1))

    h1, nrm = _prescale(dp_t, init_emb, N)
    srcT1, dstT1 = edge_tables(2, 2)
    agg1 = _spmm_kernel(N, E, 2, 160, 2)(
        h1.reshape(2 * N, 160), srcT1, dstT1)

    h2 = _layer1(agg1, nrm, W1p, b1r, N)
    srcT2, dstT2 = edge_tables(4, 5)
    agg2 = _spmm_kernel(N, E, 4, 112, 5)(
        h2.reshape(4 * N, 112), srcT2, dstT2)

    return _layer2(agg2, nrm, W2p, b2r, N)


# speedup vs baseline: 1.5421x; 1.5421x over previous
import functools

import jax
import jax.numpy as jnp
from jax import lax
from jax.experimental import pallas as pl
from jax.experimental.pallas import tpu as pltpu
from jax.experimental.pallas import tpu_sc as plsc

NCORE = 2
NS = 16
LANES = 16
EB = 80
RBLK = 2000

_SC_PARAMS = pltpu.CompilerParams(
    needs_layout_passes=False, use_tc_tiling_on_sc=False)


def _degree_kernel(N, E):
    per_tec = E // NS
    n_vec = per_tec // LANES
    mesh = plsc.VectorSubcoreMesh(core_axis_name="c", subcore_axis_name="s")

    @functools.partial(
        pl.kernel,
        mesh=mesh,
        out_type=jax.ShapeDtypeStruct((NCORE, NS, N), jnp.float32),
        compiler_params=_SC_PARAMS,
        scratch_types=[
            pltpu.VMEM((per_tec,), jnp.int32),
            pltpu.VMEM((N,), jnp.float32),
        ],
    )
    def deg_k(ei, out, ibuf, hist):
        cid = lax.axis_index("c")
        sid = lax.axis_index("s")
        pltpu.sync_copy(ei.at[cid].at[sid], ibuf)

        z16 = jnp.zeros((LANES,), jnp.float32)

        def zero_body(i, _):
            hist[pl.ds(i * LANES, LANES)] = z16
            return 0

        lax.fori_loop(0, N // LANES, zero_body, 0)

        ones16 = jnp.ones((LANES,), jnp.float32)

        def hist_body(i, _):
            idx = ibuf[pl.ds(i * LANES, LANES)]
            plsc.addupdate_scatter(hist, [idx], ones16)
            return 0

        lax.fori_loop(0, n_vec, hist_body, 0)
        pltpu.sync_copy(hist, out.at[cid].at[sid])

    return deg_k


def _spmm_kernel(N, E, C, Dc, EB2, BV):
    per_tec = E // NS
    nb = per_tec // EB2
    NBLK = nb // BV
    NBP = NBLK // 2
    assert NBP * 2 * BV * EB2 == per_tec
    cpc = C // NCORE
    rows_per_tile = N // NS
    mesh = plsc.VectorSubcoreMesh(core_axis_name="c", subcore_axis_name="s")

    @functools.partial(
        pl.kernel,
        mesh=mesh,
        out_type=jax.ShapeDtypeStruct((C, N, Dc), jnp.float32),
        compiler_params=_SC_PARAMS,
        scratch_types=[
            pltpu.VMEM((EB2, Dc), jnp.float32),
            pltpu.VMEM((EB2, Dc), jnp.float32),
            pltpu.VMEM((BV, 2, EB2), jnp.int32),
            pltpu.VMEM((BV, 2, EB2), jnp.int32),
            pltpu.SemaphoreType.DMA,
            pltpu.SemaphoreType.DMA,
            pltpu.SemaphoreType.DMA,
            pltpu.SemaphoreType.DMA,
            pltpu.VMEM_SHARED((N, Dc), jnp.float32),
        ],
    )
    def spmm_k(tab, et, out, rows0, rows1, ib0, ib1, g0, g1, i0, i1, acc):
        rows = (rows0, rows1)
        ibig = (ib0, ib1)
        gsem = (g0, g1)
        isem = (i0, i1)
        cid = lax.axis_index("c")
        sid = lax.axis_index("s")
        etk = et.at[sid]
        row0 = sid * rows_per_tile
        z16 = jnp.zeros((LANES,), jnp.float32)

        def offset_block(q, koff):
            for v in range(BV):
                for t in range(EB2 // LANES):
                    sl = pl.ds(t * LANES, LANES)
                    ibig[q][v, 0, sl] = ibig[q][v, 0, sl] + koff

        def fire_gather(q, v, b):
            pltpu.async_copy(tab.at[ibig[q].at[v].at[0]], rows[b], gsem[b])

        def wait_gather(q, v, b):
            pltpu.make_async_copy(
                tab.at[ibig[q].at[v].at[0]], rows[b], gsem[b]).wait()

        for ci in range(cpc):
            k = cid + NCORE * ci
            koff = k * N

            pltpu.sync_copy(etk.at[0], ibig[0])
            offset_block(0, koff)
            pltpu.async_copy(etk.at[1], ibig[1], isem[1])

            def zfill(r, _):
                for t in range(Dc // LANES):
                    rows0[r, pl.ds(t * LANES, LANES)] = z16
                return 0

            lax.fori_loop(0, EB2, zfill, 0)
            nfull = rows_per_tile // EB2
            rem = rows_per_tile - nfull * EB2
            for j in range(nfull):
                pltpu.sync_copy(rows0, acc.at[pl.ds(row0 + j * EB2, EB2)])
            if rem:
                pltpu.sync_copy(rows0.at[pl.ds(0, rem)],
                                acc.at[pl.ds(row0 + nfull * EB2, rem)])
            plsc.subcore_barrier()

            fire_gather(0, 0, 0)
            fire_gather(0, 1, 1)

            def pair_body(bp, _):
                for q in range(2):
                    m = 2 * bp + q
                    for v in range(BV):
                        b = (q + v) % 2
                        wait_gather(q, v, b)
                        pltpu.sync_copy(
                            rows[b], acc.at[ibig[q].at[v].at[1]], add=True)
                        if v < BV - 2:
                            fire_gather(q, v + 2, b)
                        elif v == BV - 2:
                            def arm(b=b, q=q, m=m):
                                pltpu.make_async_copy(
                                    etk.at[m + 1], ibig[1 - q],
                                    isem[1 - q]).wait()
                                offset_block(1 - q, koff)
                                fire_gather(1 - q, 0, b)
                            if q == 0:
                                arm()
                            else:
                                pl.when(bp < NBP - 1)(arm)
                        else:
                            def tail(b=b, q=q, m=m):
                                fire_gather(1 - q, 1, b)
                            def nxt(q=q, m=m):
                                pltpu.async_copy(
                                    etk.at[m + 2], ibig[q], isem[q])
                            if q == 0:
                                tail()
                            else:
                                pl.when(bp < NBP - 1)(tail)
                            pl.when(bp < NBP - 1)(nxt)
                return 0

            lax.fori_loop(0, NBP, pair_body, 0)
            plsc.subcore_barrier()
            sl = pl.ds(row0, rows_per_tile)
            pltpu.sync_copy(acc.at[sl], out.at[k].at[sl])

    return spmm_k


def _prescale(dp_t, emb, N):
    nblk = N // RBLK

    def body(dp_ref, emb_ref, h1_ref, nrm_ref):
        deg = jnp.sum(dp_ref[...], axis=2)
        nrm = jnp.where(deg > 0, lax.rsqrt(deg), 0.0)
        nrm_ref[...] = nrm
        h = emb_ref[...] * nrm[:, 0:1]
        h1_ref[0] = h[:, :160]
        h1_ref[1] = jnp.concatenate(
            [h[:, 160:300], jnp.zeros((RBLK, 20), jnp.float32)], axis=1)

    return pl.pallas_call(
        body,
        grid=(nblk,),
        in_specs=[
            pl.BlockSpec((RBLK, NCORE, NS), lambda i: (i, 0, 0)),
            pl.BlockSpec((RBLK, 300), lambda i: (i, 0)),
        ],
        out_specs=[
            pl.BlockSpec((2, RBLK, 160), lambda i: (0, i, 0)),
            pl.BlockSpec((RBLK, 2), lambda i: (i, 0)),
        ],
        out_shape=[
            jax.ShapeDtypeStruct((2, N, 160), jnp.float32),
            jax.ShapeDtypeStruct((N, 2), jnp.float32),
        ],
    )(dp_t, emb)


def _layer1(agg1, nrm, W1p, b1, N):
    nblk = N // RBLK

    def body(a_ref, n_ref, w_ref, b_ref, o_ref):
        x = jnp.concatenate([a_ref[0], a_ref[1]], axis=1)
        nb2 = n_ref[...]
        x = x * nb2[:, 1:2]
        y = jnp.dot(x, w_ref[...], preferred_element_type=jnp.float32)
        y = y + b_ref[...]
        y = jnp.where(y >= 0.0, y, 0.2 * y)
        y = y * nb2[:, 0:1]
        for kk in range(3):
            o_ref[kk] = y[:, 112 * kk:112 * (kk + 1)]
        o_ref[3] = jnp.concatenate(
            [y[:, 336:400], jnp.zeros((RBLK, 48), jnp.float32)], axis=1)

    return pl.pallas_call(
        body,
        grid=(nblk,),
        in_specs=[
            pl.BlockSpec((2, RBLK, 160), lambda i: (0, i, 0)),
            pl.BlockSpec((RBLK, 2), lambda i: (i, 0)),
            pl.BlockSpec((320, 400), lambda i: (0, 0)),
            pl.BlockSpec((1, 400), lambda i: (0, 0)),
        ],
        out_specs=pl.BlockSpec((4, RBLK, 112), lambda i: (0, i, 0)),
        out_shape=jax.ShapeDtypeStruct((4, N, 112), jnp.float32),
    )(agg1, nrm, W1p, b1)


def _layer2(agg2, nrm, W2p, b2, N):
    nblk = N // RBLK

    def body(a_ref, n_ref, w_ref, b_ref, o_ref):
        x = jnp.concatenate([a_ref[kk] for kk in range(4)], axis=1)
        x = x * n_ref[:, 1:2]
        o_ref[...] = (
            jnp.dot(x, w_ref[...], preferred_element_type=jnp.float32)
            + b_ref[...])

    return pl.pallas_call(
        body,
        grid=(nblk,),
        in_specs=[
            pl.BlockSpec((4, RBLK, 112), lambda i: (0, i, 0)),
            pl.BlockSpec((RBLK, 2), lambda i: (i, 0)),
            pl.BlockSpec((448, 512), lambda i: (0, 0)),
            pl.BlockSpec((1, 512), lambda i: (0, 0)),
        ],
        out_specs=pl.BlockSpec((RBLK, 512), lambda i: (i, 0)),
        out_shape=jax.ShapeDtypeStruct((N, 512), jnp.float32),
    )(agg2, nrm, W2p, b2)


def kernel(init_emb, W1, b1, W2, b2, edge_index):
    N = init_emb.shape[0]
    E = edge_index.shape[1]

    per_tec = E // NS
    ei3d = edge_index.reshape(2, NS, per_tec)

    def edge_blocks(eb, bv):
        nbb = per_tec // eb
        return jnp.stack(
            [edge_index[0].reshape(NS, nbb // bv, bv, eb),
             edge_index[1].reshape(NS, nbb // bv, bv, eb)], axis=3)

    W1p = jnp.concatenate([W1, jnp.zeros((20, 400), jnp.float32)], axis=0)
    W2p = jnp.concatenate([W2, jnp.zeros((48, 512), jnp.float32)], axis=0)
    b1r = b1.reshape(1, 400)
    b2r = b2.reshape(1, 512)

    deg_parts = _degree_kernel(N, E)(ei3d)
    dp_t = jnp.transpose(deg_parts, (2, 0, 1))

    h1, nrm = _prescale(dp_t, init_emb, N)
    agg1 = _spmm_kernel(N, E, 2, 160, 80, 5)(
        h1.reshape(2 * N, 160), edge_blocks(80, 5))

    h2 = _layer1(agg1, nrm, W1p, b1r, N)
    agg2 = _spmm_kernel(N, E, 4, 112, 80, 25)(
        h2.reshape(4 * N, 112), edge_blocks(80, 25))

    return _layer2(agg2, nrm, W2p, b2r, N)


# speedup vs baseline: 1.5498x; 1.0049x over previous
import functools

import jax
import jax.numpy as jnp
from jax import lax
from jax.experimental import pallas as pl
from jax.experimental.pallas import tpu as pltpu
from jax.experimental.pallas import tpu_sc as plsc

NCORE = 2
NS = 16
LANES = 16
EB = 80
RBLK = 2000

_SC_PARAMS = pltpu.CompilerParams(
    needs_layout_passes=False, use_tc_tiling_on_sc=False)


def _degree_kernel(N, E):
    per_tec = E // NS
    n_vec = per_tec // LANES
    mesh = plsc.VectorSubcoreMesh(core_axis_name="c", subcore_axis_name="s")

    @functools.partial(
        pl.kernel,
        mesh=mesh,
        out_type=jax.ShapeDtypeStruct((NCORE, NS, N), jnp.float32),
        compiler_params=_SC_PARAMS,
        scratch_types=[
            pltpu.VMEM((per_tec,), jnp.int32),
            pltpu.VMEM((N,), jnp.float32),
        ],
    )
    def deg_k(ei, out, ibuf, hist):
        cid = lax.axis_index("c")
        sid = lax.axis_index("s")
        pltpu.sync_copy(ei.at[cid].at[sid], ibuf)

        z16 = jnp.zeros((LANES,), jnp.float32)

        def zero_body(i, _):
            hist[pl.ds(i * LANES, LANES)] = z16
            return 0

        lax.fori_loop(0, N // LANES, zero_body, 0)

        ones16 = jnp.ones((LANES,), jnp.float32)

        def hist_body(i, _):
            idx = ibuf[pl.ds(i * LANES, LANES)]
            plsc.addupdate_scatter(hist, [idx], ones16)
            return 0

        lax.fori_loop(0, n_vec, hist_body, 0)
        pltpu.sync_copy(hist, out.at[cid].at[sid])

    return deg_k


def _spmm_kernel(N, E, C, Dc, EB2, BV):
    per_tec = E // NS
    nb = per_tec // EB2
    NBLK = nb // BV
    NBP = NBLK // 2
    assert NBP * 2 * BV * EB2 == per_tec
    cpc = C // NCORE
    rows_per_tile = N // NS
    mesh = plsc.VectorSubcoreMesh(core_axis_name="c", subcore_axis_name="s")

    @functools.partial(
        pl.kernel,
        mesh=mesh,
        out_type=jax.ShapeDtypeStruct((C, N, Dc), jnp.float32),
        compiler_params=_SC_PARAMS,
        scratch_types=[
            pltpu.VMEM((EB2, Dc), jnp.float32),
            pltpu.VMEM((EB2, Dc), jnp.float32),
            pltpu.VMEM((BV, 2, EB2), jnp.int32),
            pltpu.VMEM((BV, 2, EB2), jnp.int32),
            pltpu.SemaphoreType.DMA,
            pltpu.SemaphoreType.DMA,
            pltpu.SemaphoreType.DMA,
            pltpu.SemaphoreType.DMA,
            pltpu.VMEM_SHARED((N, Dc), jnp.float32),
        ],
    )
    def spmm_k(tab, et, out, rows0, rows1, ib0, ib1, g0, g1, i0, i1, acc):
        rows = (rows0, rows1)
        ibig = (ib0, ib1)
        gsem = (g0, g1)
        isem = (i0, i1)
        cid = lax.axis_index("c")
        sid = lax.axis_index("s")
        etk = et.at[sid]
        row0 = sid * rows_per_tile
        z16 = jnp.zeros((LANES,), jnp.float32)

        def offset_block(q, koff):
            for v in range(BV):
                for t in range(EB2 // LANES):
                    sl = pl.ds(t * LANES, LANES)
                    ibig[q][v, 0, sl] = ibig[q][v, 0, sl] + koff

        def fire_gather(q, v, b):
            pltpu.async_copy(tab.at[ibig[q].at[v].at[0]], rows[b], gsem[b])

        def wait_gather(q, v, b):
            pltpu.make_async_copy(
                tab.at[ibig[q].at[v].at[0]], rows[b], gsem[b]).wait()

        for ci in range(cpc):
            k = cid + NCORE * ci
            koff = k * N

            pltpu.sync_copy(etk.at[0], ibig[0])
            offset_block(0, koff)
            pltpu.async_copy(etk.at[1], ibig[1], isem[1])

            def zfill(r, _):
                for t in range(Dc // LANES):
                    rows0[r, pl.ds(t * LANES, LANES)] = z16
                return 0

            lax.fori_loop(0, EB2, zfill, 0)
            nfull = rows_per_tile // EB2
            rem = rows_per_tile - nfull * EB2
            for j in range(nfull):
                pltpu.sync_copy(rows0, acc.at[pl.ds(row0 + j * EB2, EB2)])
            if rem:
                pltpu.sync_copy(rows0.at[pl.ds(0, rem)],
                                acc.at[pl.ds(row0 + nfull * EB2, rem)])
            plsc.subcore_barrier()

            fire_gather(0, 0, 0)
            fire_gather(0, 1, 1)

            def pair_body(bp, _):
                for q in range(2):
                    m = 2 * bp + q
                    for v in range(BV):
                        b = (q + v) % 2
                        wait_gather(q, v, b)
                        pltpu.sync_copy(
                            rows[b], acc.at[ibig[q].at[v].at[1]], add=True)
                        if v < BV - 2:
                            fire_gather(q, v + 2, b)
                        elif v == BV - 2:
                            def arm(b=b, q=q, m=m):
                                pltpu.make_async_copy(
                                    etk.at[m + 1], ibig[1 - q],
                                    isem[1 - q]).wait()
                                offset_block(1 - q, koff)
                                fire_gather(1 - q, 0, b)
                            if q == 0:
                                arm()
                            else:
                                pl.when(bp < NBP - 1)(arm)
                        else:
                            def tail(b=b, q=q, m=m):
                                fire_gather(1 - q, 1, b)
                            def nxt(q=q, m=m):
                                pltpu.async_copy(
                                    etk.at[m + 2], ibig[q], isem[q])
                            if q == 0:
                                tail()
                            else:
                                pl.when(bp < NBP - 1)(tail)
                            pl.when(bp < NBP - 1)(nxt)
                return 0

            lax.fori_loop(0, NBP, pair_body, 0)
            plsc.subcore_barrier()
            sl = pl.ds(row0, rows_per_tile)
            pltpu.sync_copy(acc.at[sl], out.at[k].at[sl])

    return spmm_k


def _prescale(dp_t, emb, N):
    nblk = N // RBLK

    def body(dp_ref, emb_ref, h1_ref, nrm_ref):
        deg = jnp.sum(dp_ref[...], axis=2)
        nrm = jnp.where(deg > 0, lax.rsqrt(deg), 0.0)
        nrm_ref[...] = nrm
        h = emb_ref[...] * nrm[:, 0:1]
        h1_ref[0] = h[:, :160]
        h1_ref[1] = jnp.concatenate(
            [h[:, 160:300], jnp.zeros((RBLK, 20), jnp.float32)], axis=1)

    return pl.pallas_call(
        body,
        grid=(nblk,),
        in_specs=[
            pl.BlockSpec((RBLK, NCORE, NS), lambda i: (i, 0, 0)),
            pl.BlockSpec((RBLK, 300), lambda i: (i, 0)),
        ],
        out_specs=[
            pl.BlockSpec((2, RBLK, 160), lambda i: (0, i, 0)),
            pl.BlockSpec((RBLK, 2), lambda i: (i, 0)),
        ],
        out_shape=[
            jax.ShapeDtypeStruct((2, N, 160), jnp.float32),
            jax.ShapeDtypeStruct((N, 2), jnp.float32),
        ],
    )(dp_t, emb)


def _layer1(agg1, nrm, W1p, b1, N):
    nblk = N // RBLK

    def body(a_ref, n_ref, w_ref, b_ref, o_ref):
        x = jnp.concatenate([a_ref[0], a_ref[1]], axis=1)
        nb2 = n_ref[...]
        x = x * nb2[:, 1:2]
        y = jnp.dot(x, w_ref[...], preferred_element_type=jnp.float32)
        y = y + b_ref[...]
        y = jnp.where(y >= 0.0, y, 0.2 * y)
        y = y * nb2[:, 0:1]
        for kk in range(3):
            o_ref[kk] = y[:, 112 * kk:112 * (kk + 1)]
        o_ref[3] = jnp.concatenate(
            [y[:, 336:400], jnp.zeros((RBLK, 48), jnp.float32)], axis=1)

    return pl.pallas_call(
        body,
        grid=(nblk,),
        in_specs=[
            pl.BlockSpec((2, RBLK, 160), lambda i: (0, i, 0)),
            pl.BlockSpec((RBLK, 2), lambda i: (i, 0)),
            pl.BlockSpec((320, 400), lambda i: (0, 0)),
            pl.BlockSpec((1, 400), lambda i: (0, 0)),
        ],
        out_specs=pl.BlockSpec((4, RBLK, 112), lambda i: (0, i, 0)),
        out_shape=jax.ShapeDtypeStruct((4, N, 112), jnp.float32),
    )(agg1, nrm, W1p, b1)


def _layer2(agg2, nrm, W2p, b2, N):
    nblk = N // RBLK

    def body(a_ref, n_ref, w_ref, b_ref, o_ref):
        x = jnp.concatenate([a_ref[kk] for kk in range(4)], axis=1)
        x = x * n_ref[:, 1:2]
        o_ref[...] = (
            jnp.dot(x, w_ref[...], preferred_element_type=jnp.float32)
            + b_ref[...])

    return pl.pallas_call(
        body,
        grid=(nblk,),
        in_specs=[
            pl.BlockSpec((4, RBLK, 112), lambda i: (0, i, 0)),
            pl.BlockSpec((RBLK, 2), lambda i: (i, 0)),
            pl.BlockSpec((448, 512), lambda i: (0, 0)),
            pl.BlockSpec((1, 512), lambda i: (0, 0)),
        ],
        out_specs=pl.BlockSpec((RBLK, 512), lambda i: (i, 0)),
        out_shape=jax.ShapeDtypeStruct((N, 512), jnp.float32),
    )(agg2, nrm, W2p, b2)


def kernel(init_emb, W1, b1, W2, b2, edge_index):
    N = init_emb.shape[0]
    E = edge_index.shape[1]

    per_tec = E // NS
    ei3d = edge_index.reshape(2, NS, per_tec)

    def edge_blocks(eb, bv):
        nbb = per_tec // eb
        return jnp.stack(
            [edge_index[0].reshape(NS, nbb // bv, bv, eb),
             edge_index[1].reshape(NS, nbb // bv, bv, eb)], axis=3)

    W1p = jnp.concatenate([W1, jnp.zeros((20, 400), jnp.float32)], axis=0)
    W2p = jnp.concatenate([W2, jnp.zeros((48, 512), jnp.float32)], axis=0)
    b1r = b1.reshape(1, 400)
    b2r = b2.reshape(1, 512)

    deg_parts = _degree_kernel(N, E)(ei3d)
    dp_t = jnp.transpose(deg_parts, (2, 0, 1))

    h1, nrm = _prescale(dp_t, init_emb, N)
    agg1 = _spmm_kernel(N, E, 2, 160, 80, 5)(
        h1.reshape(2 * N, 160), edge_blocks(80, 5))

    h2 = _layer1(agg1, nrm, W1p, b1r, N)
    agg2 = _spmm_kernel(N, E, 4, 112, 80, 5)(
        h2.reshape(4 * N, 112), edge_blocks(80, 5))

    return _layer2(agg2, nrm, W2p, b2r, N)
